# baseline (speedup 1.0000x reference)
import functools

import jax
import jax.numpy as jnp
from jax import lax
from jax.experimental import pallas as pl
from jax.experimental.pallas import tpu as pltpu
from jax.experimental.pallas import tpu_sc as plsc

_NC = 2
_NS = 16
_NW = _NC * _NS
_LANES = 16
_BLK = 32


def _sc_add(x2, table):
    rows, d = x2.shape
    seq_len = table.shape[0]
    rows_per_w = rows // _NW
    n_blocks = rows_per_w // _BLK
    lanes_per_row = d // _LANES

    mesh = plsc.VectorSubcoreMesh(core_axis_name="c", subcore_axis_name="s")

    @functools.partial(
        pl.kernel,
        mesh=mesh,
        out_type=jax.ShapeDtypeStruct((rows, d), jnp.float32),
        scratch_types=[
            pltpu.VMEM((_BLK, d), jnp.float32),
            pltpu.VMEM((_BLK, d), jnp.float32),
        ],
    )
    def k(x_hbm, tab_hbm, out_hbm, xbuf, tbuf):
        wid = lax.axis_index("s") * _NC + lax.axis_index("c")
        base = wid * rows_per_w
        tab_base = base % seq_len

        def block(i, carry):
            row = base + i * _BLK
            trow = tab_base + i * _BLK
            pltpu.sync_copy(x_hbm.at[pl.ds(row, _BLK)], xbuf)
            pltpu.sync_copy(tab_hbm.at[pl.ds(trow, _BLK)], tbuf)

            def row_add(r, c2):
                for c in range(lanes_per_row):
                    sl = pl.ds(c * _LANES, _LANES)
                    xbuf[r, sl] = xbuf[r, sl] + tbuf[r, sl]
                return c2

            lax.fori_loop(0, _BLK, row_add, 0)
            pltpu.sync_copy(xbuf, out_hbm.at[pl.ds(row, _BLK)])
            return carry

        lax.fori_loop(0, n_blocks, block, 0)

    return k(x2, table)


def kernel(x, embedding_table):
    batch, seq_len, d_model = x.shape
    table = embedding_table[:seq_len]
    x2 = x.reshape(batch * seq_len, d_model)
    out = _sc_add(x2, table)
    return out.reshape(batch, seq_len, d_model)


# speedup vs baseline: 1.4708x; 1.4708x over previous
import functools

import jax
import jax.numpy as jnp
from jax import lax
from jax.experimental import pallas as pl
from jax.experimental.pallas import tpu as pltpu
from jax.experimental.pallas import tpu_sc as plsc

_NC = 2
_NS = 16
_NW = _NC * _NS
_LANES = 16
_BLK = 16
_NBUF = 2


def _sc_add(x2, table):
    rows, d = x2.shape
    seq_len = table.shape[0]
    rows_per_w = rows // _NW
    n_blocks = rows_per_w // _BLK
    lanes_per_row = d // _LANES

    mesh = plsc.VectorSubcoreMesh(core_axis_name="c", subcore_axis_name="s")

    @functools.partial(
        pl.kernel,
        mesh=mesh,
        out_type=jax.ShapeDtypeStruct((rows, d), jnp.float32),
        scratch_types=[
            pltpu.VMEM((_NBUF, _BLK, d), jnp.float32),
            pltpu.VMEM((_NBUF, _BLK, d), jnp.float32),
            pltpu.VMEM((_NBUF, _BLK, d), jnp.float32),
            pltpu.SemaphoreType.DMA,
            pltpu.SemaphoreType.DMA,
            pltpu.SemaphoreType.DMA,
            pltpu.SemaphoreType.DMA,
        ],
    )
    def k(x_hbm, tab_hbm, out_hbm, xbuf, tbuf, obuf, isem0, isem1, osem0, osem1):
        wid = lax.axis_index("s") * _NC + lax.axis_index("c")
        base = wid * rows_per_w
        tab_base = base % seq_len
        isems = (isem0, isem1)
        osems = (osem0, osem1)

        def start_in(g, b):
            pltpu.async_copy(
                x_hbm.at[pl.ds(base + g * _BLK, _BLK)], xbuf.at[b], isems[b])
            pltpu.async_copy(
                tab_hbm.at[pl.ds(tab_base + g * _BLK, _BLK)], tbuf.at[b], isems[b])

        def wait_in(g, b):
            pltpu.make_async_copy(
                x_hbm.at[pl.ds(base + g * _BLK, _BLK)], xbuf.at[b], isems[b]).wait()
            pltpu.make_async_copy(
                tab_hbm.at[pl.ds(tab_base + g * _BLK, _BLK)], tbuf.at[b], isems[b]).wait()

        def start_out(g, b):
            pltpu.async_copy(
                obuf.at[b], out_hbm.at[pl.ds(base + g * _BLK, _BLK)], osems[b])

        def wait_out(g, b):
            pltpu.make_async_copy(
                obuf.at[b], out_hbm.at[pl.ds(base + g * _BLK, _BLK)], osems[b]).wait()

        def compute(b):
            def row_add(r, c2):
                for c in range(lanes_per_row):
                    sl = pl.ds(c * _LANES, _LANES)
                    obuf[b, r, sl] = xbuf[b, r, sl] + tbuf[b, r, sl]
                return c2

            lax.fori_loop(0, _BLK, row_add, 0)

        for b in range(_NBUF):
            start_in(b, b)

        def ring_step(g2, carry):
            for b in range(_NBUF):
                g = g2 * _NBUF + b
                wait_in(g, b)

                @pl.when(g2 > 0)
                def _():
                    wait_out(g - _NBUF, b)

                compute(b)
                start_out(g, b)

                @pl.when(g + _NBUF < n_blocks)
                def _():
                    start_in(g + _NBUF, b)
            return carry

        lax.fori_loop(0, n_blocks // _NBUF, ring_step, 0)

        for b in range(_NBUF):
            wait_out(n_blocks - _NBUF + b, b)

    return k(x2, table)


def kernel(x, embedding_table):
    batch, seq_len, d_model = x.shape
    table = embedding_table[:seq_len]
    x2 = x.reshape(batch * seq_len, d_model)
    out = _sc_add(x2, table)
    return out.reshape(batch, seq_len, d_model)


# speedup vs baseline: 3.3134x; 2.2528x over previous
import jax
import jax.numpy as jnp
from jax.experimental import pallas as pl
from jax.experimental.pallas import tpu as pltpu

_SEQ_BLOCK = 2048


def _add_block(x_ref, tab_ref, o_ref):
    o_ref[...] = x_ref[...] + tab_ref[...]


def kernel(x, embedding_table):
    batch, seq_len, d_model = x.shape
    table = embedding_table[:seq_len]

    blk = _SEQ_BLOCK
    if seq_len % blk != 0:
        blk = seq_len
    grid = (seq_len // blk, batch)

    return pl.pallas_call(
        _add_block,
        grid=grid,
        in_specs=[
            pl.BlockSpec((1, blk, d_model), lambda i, b: (b, i, 0)),
            pl.BlockSpec((blk, d_model), lambda i, b: (i, 0)),
        ],
        out_specs=pl.BlockSpec((1, blk, d_model), lambda i, b: (b, i, 0)),
        out_shape=jax.ShapeDtypeStruct((batch, seq_len, d_model), x.dtype),
        compiler_params=pltpu.CompilerParams(
            vmem_limit_bytes=60 * 1024 * 1024,
        ),
    )(x, table)
